# baseline (speedup 1.0000x reference)
import functools

import jax
import jax.numpy as jnp
from jax import lax
from jax.experimental import pallas as pl
from jax.experimental.pallas import tpu as pltpu
from jax.experimental.pallas import tpu_sc as plsc

NC = 2
NS = 16
NW = NC * NS

LANES = 16
BB = 128
HC = 20
IDX_RING = 3 * HC


def _build(n_b, n_h, d_w, d_f):
  assert n_b == NW * BB and n_h % HC == 0 and n_h >= 2 * HC
  n_blk = n_h // HC
  tr_w = d_w // 8
  tr_o = (d_w + d_f) // 8
  n_bt = n_b // BB
  mesh = plsc.VectorSubcoreMesh(
      core_axis_name="c", subcore_axis_name="s",
      num_cores=NC, num_subcores=NS)

  @functools.partial(
      pl.kernel,
      out_type=jax.ShapeDtypeStruct((n_h, tr_o, n_bt, 8, BB), jnp.float32),
      mesh=mesh,
      scratch_types=[
          pltpu.VMEM((IDX_RING, BB), jnp.int32),
          pltpu.VMEM((IDX_RING, BB), jnp.int32),
          pltpu.VMEM((2 * BB, d_w), jnp.float32),
          pltpu.VMEM((2, tr_o, 8, BB), jnp.float32),
          pltpu.VMEM((4, d_f), jnp.float32),
          pltpu.SemaphoreType.DMA,
          pltpu.SemaphoreType.DMA,
          pltpu.SemaphoreType.DMA,
      ],
      compiler_params=pltpu.CompilerParams(
          use_tc_tiling_on_sc=False, needs_layout_passes=False),
  )
  def body(xt_hbm, yt_hbm, wv_hbm, ft_hbm, out_hbm,
           xi, yi, wbuf, cbuf, fvm, sem_idx, sem_g, sem_wr):
    wid = lax.axis_index("s") * NC + lax.axis_index("c")
    b0 = wid * BB
    iota = jnp.arange(LANES, dtype=jnp.int32)

    def idx_copies(blk):
      r0 = blk * HC
      s0 = (blk % 3) * HC
      return (
          pltpu.make_async_copy(
              xt_hbm.at[pl.ds(r0, HC), pl.ds(b0, BB)],
              xi.at[pl.ds(s0, HC)], sem_idx),
          pltpu.make_async_copy(
              yt_hbm.at[pl.ds(r0, HC), pl.ds(b0, BB)],
              yi.at[pl.ds(s0, HC)], sem_idx),
      )

    def gather_copy(h):
      return pltpu.make_async_copy(
          wv_hbm.at[xi.at[h % IDX_RING]],
          wbuf.at[pl.ds((h % 2) * BB, BB)], sem_g)

    def write_copy(h):
      return pltpu.make_async_copy(
          cbuf.at[h % 2], out_hbm.at[h, :, wid], sem_wr)

    def start(cs):
      for c in cs:
        c.start()

    def wait(cs):
      for c in cs:
        c.wait()

    def transpose(h):
      slot = h % 2
      wb = slot * BB
      hr = h % IDX_RING

      def m_loop(m, carry):
        rows = wb + m * LANES + iota
        dst = pl.ds(m * LANES, LANES)
        for c in range(d_w):
          col = jnp.full((LANES,), c, dtype=jnp.int32)
          v = plsc.load_gather(wbuf, [rows, col])
          cbuf[slot, c // 8, c % 8, dst] = v
        yv = yi[hr, dst]
        for c in range(d_f):
          col = jnp.full((LANES,), c, dtype=jnp.int32)
          v = plsc.load_gather(fvm, [yv, col])
          cbuf[slot, tr_w + c // 8, c % 8, dst] = v
        return carry

      lax.fori_loop(0, BB // LANES, m_loop, 0)

    pltpu.sync_copy(ft_hbm, fvm)
    start(idx_copies(0))
    start(idx_copies(1))
    wait(idx_copies(0))
    gather_copy(0).start()
    for h in (0, 1):
      gather_copy(h).wait()
      gather_copy(h + 1).start()
      transpose(h)
      write_copy(h).start()

    def h_body(h, carry):
      nxt = h + 1

      @pl.when(lax.rem(nxt, HC) == 0)
      def _():
        blk = nxt // HC
        wait(idx_copies(blk))

        @pl.when(blk + 1 < n_blk)
        def _():
          start(idx_copies(blk + 1))

      write_copy(h - 2).wait()
      gather_copy(h).wait()
      gather_copy(nxt).start()
      transpose(h)
      write_copy(h).start()
      return carry

    lax.fori_loop(2, n_h - 1, h_body, 0)

    h = n_h - 1
    write_copy(h - 2).wait()
    gather_copy(h).wait()
    transpose(h)
    write_copy(h).start()
    write_copy(h - 1).wait()
    write_copy(h).wait()

  return body


def kernel(x, y, word_vectors, f_table):
  b, h = x.shape
  d_w = word_vectors.shape[1]
  d_f = f_table.shape[1]
  xt = jnp.transpose(x).astype(jnp.int32)
  yt = jnp.transpose(y).astype(jnp.int32)
  body = _build(b, h, d_w, d_f)
  out5 = body(xt, yt, word_vectors, f_table)
  return jnp.transpose(out5, (2, 4, 0, 1, 3)).reshape(b, h, d_w + d_f)


# speedup vs baseline: 1.5580x; 1.5580x over previous
import functools

import jax
import jax.numpy as jnp
from jax import lax
from jax.experimental import pallas as pl
from jax.experimental.pallas import tpu as pltpu
from jax.experimental.pallas import tpu_sc as plsc

NC = 2
NS = 16
NW = NC * NS

LANES = 16
BB = 128
HC = 20
IDX_RING = 3 * HC


def _build(n_b, n_h, d_w, d_f):
  assert n_b == NW * BB and n_h % HC == 0 and n_h >= 2 * HC
  n_blk = n_h // HC
  tr_w = d_w // 8
  tr_o = (d_w + d_f) // 8
  n_bt = n_b // BB
  mesh = plsc.VectorSubcoreMesh(
      core_axis_name="c", subcore_axis_name="s",
      num_cores=NC, num_subcores=NS)

  @functools.partial(
      pl.kernel,
      out_type=jax.ShapeDtypeStruct((n_h, tr_o, n_bt, 8, BB), jnp.float32),
      mesh=mesh,
      scratch_types=[
          pltpu.VMEM((IDX_RING, BB), jnp.int32),
          pltpu.VMEM((IDX_RING, BB), jnp.int32),
          pltpu.VMEM((2 * BB, d_w), jnp.float32),
          pltpu.VMEM((2, tr_o, 8, BB), jnp.float32),
          pltpu.VMEM((4, d_f), jnp.float32),
          pltpu.SemaphoreType.DMA,
          pltpu.SemaphoreType.DMA,
          pltpu.SemaphoreType.DMA,
      ],
      compiler_params=pltpu.CompilerParams(
          use_tc_tiling_on_sc=False, needs_layout_passes=False),
  )
  def body(xt_hbm, yt_hbm, wv_hbm, ft_hbm, out_hbm,
           xi, yi, wbuf, cbuf, fvm, sem_idx, sem_g, sem_wr):
    wid = lax.axis_index("s") * NC + lax.axis_index("c")
    b0 = wid * BB
    iota = jnp.arange(LANES, dtype=jnp.int32)

    def idx_copies(blk):
      r0 = blk * HC
      s0 = (blk % 3) * HC
      return (
          pltpu.make_async_copy(
              xt_hbm.at[pl.ds(r0, HC), pl.ds(b0, BB)],
              xi.at[pl.ds(s0, HC)], sem_idx),
          pltpu.make_async_copy(
              yt_hbm.at[pl.ds(r0, HC), pl.ds(b0, BB)],
              yi.at[pl.ds(s0, HC)], sem_idx),
      )

    def gather_copy(h):
      return pltpu.make_async_copy(
          wv_hbm.at[xi.at[h % IDX_RING]],
          wbuf.at[pl.ds((h % 2) * BB, BB)], sem_g)

    def write_copy(h):
      return pltpu.make_async_copy(
          cbuf.at[h % 2], out_hbm.at[h, :, wid], sem_wr)

    def start(cs):
      for c in cs:
        c.start()

    def wait(cs):
      for c in cs:
        c.wait()

    def transpose(h):
      slot = h % 2
      wb = slot * BB
      hr = h % IDX_RING
      n_m = BB // LANES
      rows_m = [wb + m * LANES + iota for m in range(n_m)]
      yv_m = [yi[hr, pl.ds(m * LANES, LANES)] for m in range(n_m)]

      @plsc.parallel_loop(0, d_w, unroll=4)
      def _(c):
        tr = c // 8
        r = lax.rem(c, 8)
        for m in range(n_m):
          col = jnp.full((LANES,), 0, dtype=jnp.int32) + c
          v = plsc.load_gather(wbuf, [rows_m[m], col])
          cbuf[slot, tr, r, pl.ds(m * LANES, LANES)] = v

      @plsc.parallel_loop(0, d_f, unroll=4)
      def _(c):
        tr = tr_w + c // 8
        r = lax.rem(c, 8)
        for m in range(n_m):
          col = jnp.full((LANES,), 0, dtype=jnp.int32) + c
          v = plsc.load_gather(fvm, [yv_m[m], col])
          cbuf[slot, tr, r, pl.ds(m * LANES, LANES)] = v

    pltpu.sync_copy(ft_hbm, fvm)
    start(idx_copies(0))
    start(idx_copies(1))
    wait(idx_copies(0))
    gather_copy(0).start()
    for h in (0, 1):
      gather_copy(h).wait()
      gather_copy(h + 1).start()
      transpose(h)
      write_copy(h).start()

    def h_body(h, carry):
      nxt = h + 1

      @pl.when(lax.rem(nxt, HC) == 0)
      def _():
        blk = nxt // HC
        wait(idx_copies(blk))

        @pl.when(blk + 1 < n_blk)
        def _():
          start(idx_copies(blk + 1))

      write_copy(h - 2).wait()
      gather_copy(h).wait()
      gather_copy(nxt).start()
      transpose(h)
      write_copy(h).start()
      return carry

    lax.fori_loop(2, n_h - 1, h_body, 0)

    h = n_h - 1
    write_copy(h - 2).wait()
    gather_copy(h).wait()
    transpose(h)
    write_copy(h).start()
    write_copy(h - 1).wait()
    write_copy(h).wait()

  return body


def kernel(x, y, word_vectors, f_table):
  b, h = x.shape
  d_w = word_vectors.shape[1]
  d_f = f_table.shape[1]
  xt = jnp.transpose(x).astype(jnp.int32)
  yt = jnp.transpose(y).astype(jnp.int32)
  body = _build(b, h, d_w, d_f)
  out5 = body(xt, yt, word_vectors, f_table)
  return jnp.transpose(out5, (2, 4, 0, 1, 3)).reshape(b, h, d_w + d_f)


# speedup vs baseline: 2.4307x; 1.5601x over previous
import functools

import jax
import jax.numpy as jnp
from jax import lax
from jax.experimental import pallas as pl
from jax.experimental.pallas import tpu as pltpu
from jax.experimental.pallas import tpu_sc as plsc

NC = 2
NS = 16
NW = NC * NS

LANES = 16
BB = 128
HC = 20
IDX_RING = 3 * HC


def _build(n_b, n_h, d_w, d_f):
  assert n_b == NW * BB and n_h % HC == 0 and n_h >= 2 * HC
  n_blk = n_h // HC
  tr_w = d_w // 8
  tr_o = (d_w + d_f) // 8
  n_bt = n_b // BB
  mesh = plsc.VectorSubcoreMesh(
      core_axis_name="c", subcore_axis_name="s",
      num_cores=NC, num_subcores=NS)

  @functools.partial(
      pl.kernel,
      out_type=jax.ShapeDtypeStruct((n_h, tr_o, n_bt, 8, BB), jnp.float32),
      mesh=mesh,
      scratch_types=[
          pltpu.VMEM((IDX_RING, BB), jnp.int32),
          pltpu.VMEM((IDX_RING, BB), jnp.int32),
          pltpu.VMEM((2 * BB, d_w), jnp.float32),
          pltpu.VMEM((2, tr_o, 8, BB + 1), jnp.float32),
          pltpu.VMEM((16, d_f + 1), jnp.float32),
          pltpu.SemaphoreType.DMA,
          pltpu.SemaphoreType.DMA,
          pltpu.SemaphoreType.DMA,
      ],
      compiler_params=pltpu.CompilerParams(
          use_tc_tiling_on_sc=False, needs_layout_passes=False),
  )
  def body(xt_hbm, yt_hbm, wv_hbm, ft_hbm, out_hbm,
           xi, yi, wbuf, cbuf, fvm, sem_idx, sem_g, sem_wr):
    wid = lax.axis_index("s") * NC + lax.axis_index("c")
    b0 = wid * BB
    iota = jnp.arange(LANES, dtype=jnp.int32)
    zeros = jnp.zeros((LANES,), dtype=jnp.int32)
    rep4 = 4 * (iota & 3)
    n_cg = d_w // LANES
    trv_cg = [(cg * LANES + iota) // 8 for cg in range(n_cg)]
    rv_cg = [(cg * LANES + iota) % 8 for cg in range(n_cg)]

    def idx_copies(blk):
      r0 = blk * HC
      s0 = (blk % 3) * HC
      return (
          pltpu.make_async_copy(
              xt_hbm.at[pl.ds(r0, HC), pl.ds(b0, BB)],
              xi.at[pl.ds(s0, HC)], sem_idx),
          pltpu.make_async_copy(
              yt_hbm.at[pl.ds(r0, HC), pl.ds(b0, BB)],
              yi.at[pl.ds(s0, HC)], sem_idx),
      )

    def gather_copy(h):
      return pltpu.make_async_copy(
          wv_hbm.at[xi.at[h % IDX_RING]],
          wbuf.at[pl.ds((h % 2) * BB, BB)], sem_g)

    def write_copy(h):
      return pltpu.make_async_copy(
          cbuf.at[h % 2, :, :, pl.ds(0, BB)], out_hbm.at[h, :, wid], sem_wr)

    def start(cs):
      for c in cs:
        c.start()

    def wait(cs):
      for c in cs:
        c.wait()

    def transpose(h):
      slot = h % 2
      wb = slot * BB
      hr = h % IDX_RING
      n_m = BB // LANES
      slotv = zeros + slot
      yrow_m = [yi[hr, pl.ds(m * LANES, LANES)] + rep4 for m in range(n_m)]

      @plsc.parallel_loop(0, BB, unroll=4)
      def _(t):
        tv = zeros + t
        for cg in range(n_cg):
          v = wbuf[wb + t, pl.ds(cg * LANES, LANES)]
          plsc.store_scatter(cbuf, [slotv, trv_cg[cg], rv_cg[cg], tv], v)

      @plsc.parallel_loop(0, d_f, unroll=2)
      def _(c):
        tr = tr_w + c // 8
        r = lax.rem(c, 8)
        colv = zeros + c
        for m in range(n_m):
          vals = plsc.load_gather(fvm, [yrow_m[m], colv])
          cbuf[slot, tr, r, pl.ds(m * LANES, LANES)] = vals

    for rep in range(4):
      pltpu.sync_copy(ft_hbm, fvm.at[pl.ds(rep * 4, 4), pl.ds(0, d_f)])
    start(idx_copies(0))
    start(idx_copies(1))
    wait(idx_copies(0))
    gather_copy(0).start()
    for h in (0, 1):
      gather_copy(h).wait()
      gather_copy(h + 1).start()
      transpose(h)
      write_copy(h).start()

    def h_body(h, carry):
      nxt = h + 1

      @pl.when(lax.rem(nxt, HC) == 0)
      def _():
        blk = nxt // HC
        wait(idx_copies(blk))

        @pl.when(blk + 1 < n_blk)
        def _():
          start(idx_copies(blk + 1))

      write_copy(h - 2).wait()
      gather_copy(h).wait()
      gather_copy(nxt).start()
      transpose(h)
      write_copy(h).start()
      return carry

    lax.fori_loop(2, n_h - 1, h_body, 0)

    h = n_h - 1
    write_copy(h - 2).wait()
    gather_copy(h).wait()
    transpose(h)
    write_copy(h).start()
    write_copy(h - 1).wait()
    write_copy(h).wait()

  return body


def kernel(x, y, word_vectors, f_table):
  b, h = x.shape
  d_w = word_vectors.shape[1]
  d_f = f_table.shape[1]
  xt = jnp.transpose(x).astype(jnp.int32)
  yt = jnp.transpose(y).astype(jnp.int32)
  body = _build(b, h, d_w, d_f)
  out5 = body(xt, yt, word_vectors, f_table)
  return jnp.transpose(out5, (2, 4, 0, 1, 3)).reshape(b, h, d_w + d_f)


# speedup vs baseline: 2.4309x; 1.0001x over previous
import functools

import jax
import jax.numpy as jnp
from jax import lax
from jax.experimental import pallas as pl
from jax.experimental.pallas import tpu as pltpu
from jax.experimental.pallas import tpu_sc as plsc

NC = 2
NS = 16
NW = NC * NS

LANES = 16
BB = 128
HB = 8
HC = 40
IDX_RING = 3 * HC // HB


def _build(n_b, n_h, d_w, d_f):
  assert n_b == NW * BB and n_h % HC == 0 and n_h >= 2 * HC and HC % HB == 0
  n_blk = n_h // HC
  tr_w = d_w // 8
  tr_o = (d_w + d_f) // 8
  n_bt = n_b // BB
  mesh = plsc.VectorSubcoreMesh(
      core_axis_name="c", subcore_axis_name="s",
      num_cores=NC, num_subcores=NS)

  @functools.partial(
      pl.kernel,
      out_type=jax.ShapeDtypeStruct((n_h, tr_o, n_bt, 8, BB), jnp.float32),
      mesh=mesh,
      scratch_types=[
          pltpu.VMEM((IDX_RING, HB, BB), jnp.int32),
          pltpu.VMEM((IDX_RING, HB, BB), jnp.int32),
          pltpu.VMEM((2 * BB, d_w), jnp.float32),
          pltpu.VMEM((2, tr_o, 8, BB + 1), jnp.float32),
          pltpu.VMEM((16, d_f + 1), jnp.float32),
          pltpu.SemaphoreType.DMA,
          pltpu.SemaphoreType.DMA,
          pltpu.SemaphoreType.DMA,
      ],
      compiler_params=pltpu.CompilerParams(
          use_tc_tiling_on_sc=False, needs_layout_passes=False),
  )
  def body(xt_hbm, yt_hbm, wv_hbm, ft_hbm, out_hbm,
           xi, yi, wbuf, cbuf, fvm, sem_idx, sem_g, sem_wr):
    wid = lax.axis_index("s") * NC + lax.axis_index("c")
    b0 = wid * BB
    iota = jnp.arange(LANES, dtype=jnp.int32)
    zeros = jnp.zeros((LANES,), dtype=jnp.int32)
    rep4 = 4 * (iota & 3)
    n_cg = d_w // LANES
    trv_cg = [(cg * LANES + iota) // 8 for cg in range(n_cg)]
    rv_cg = [(cg * LANES + iota) % 8 for cg in range(n_cg)]

    hbc = HC // HB

    def idx_copies(blk):
      r0 = blk * hbc
      s0 = (blk % 3) * hbc
      return (
          pltpu.make_async_copy(
              xt_hbm.at[pl.ds(r0, hbc), wid], xi.at[pl.ds(s0, hbc)], sem_idx),
          pltpu.make_async_copy(
              yt_hbm.at[pl.ds(r0, hbc), wid], yi.at[pl.ds(s0, hbc)], sem_idx),
      )

    def gather_copy(h):
      return pltpu.make_async_copy(
          wv_hbm.at[xi.at[(h // HB) % IDX_RING, h % HB]],
          wbuf.at[pl.ds((h % 2) * BB, BB)], sem_g)

    def write_copy(h):
      return pltpu.make_async_copy(
          cbuf.at[h % 2, :, :, pl.ds(0, BB)], out_hbm.at[h, :, wid], sem_wr)

    def start(cs):
      for c in cs:
        c.start()

    def wait(cs):
      for c in cs:
        c.wait()

    def transpose(h):
      slot = h % 2
      wb = slot * BB
      n_m = BB // LANES
      slotv = zeros + slot
      yrow_m = [yi[(h // HB) % IDX_RING, h % HB, pl.ds(m * LANES, LANES)] + rep4
                for m in range(n_m)]

      @plsc.parallel_loop(0, BB, unroll=4)
      def _(t):
        tv = zeros + t
        for cg in range(n_cg):
          v = wbuf[wb + t, pl.ds(cg * LANES, LANES)]
          plsc.store_scatter(cbuf, [slotv, trv_cg[cg], rv_cg[cg], tv], v)

      @plsc.parallel_loop(0, d_f, unroll=2)
      def _(c):
        tr = tr_w + c // 8
        r = lax.rem(c, 8)
        colv = zeros + c
        for m in range(n_m):
          vals = plsc.load_gather(fvm, [yrow_m[m], colv])
          cbuf[slot, tr, r, pl.ds(m * LANES, LANES)] = vals

    for rep in range(4):
      pltpu.sync_copy(ft_hbm, fvm.at[pl.ds(rep * 4, 4), pl.ds(0, d_f)])
    start(idx_copies(0))
    start(idx_copies(1))
    wait(idx_copies(0))
    gather_copy(0).start()
    for h in (0, 1):
      gather_copy(h).wait()
      gather_copy(h + 1).start()
      transpose(h)
      write_copy(h).start()

    def h_body(h, carry):
      nxt = h + 1

      @pl.when(lax.rem(nxt, HC) == 0)
      def _():
        blk = nxt // HC
        wait(idx_copies(blk))

        @pl.when(blk + 1 < n_blk)
        def _():
          start(idx_copies(blk + 1))

      write_copy(h - 2).wait()
      gather_copy(h).wait()
      gather_copy(nxt).start()
      transpose(h)
      write_copy(h).start()
      return carry

    lax.fori_loop(2, n_h - 1, h_body, 0)

    h = n_h - 1
    write_copy(h - 2).wait()
    gather_copy(h).wait()
    transpose(h)
    write_copy(h).start()
    write_copy(h - 1).wait()
    write_copy(h).wait()

  return body


def kernel(x, y, word_vectors, f_table):
  b, h = x.shape
  d_w = word_vectors.shape[1]
  d_f = f_table.shape[1]
  def tile_view(a):
    return (jnp.transpose(a).astype(jnp.int32)
            .reshape(h // HB, HB, b // BB, BB).transpose(0, 2, 1, 3))

  body = _build(b, h, d_w, d_f)
  out5 = body(tile_view(x), tile_view(y), word_vectors, f_table)
  return jnp.transpose(out5, (2, 4, 0, 1, 3)).reshape(b, h, d_w + d_f)
